# baseline (speedup 1.0000x reference)
import dataclasses
import functools

import jax
import jax.numpy as jnp
from jax import lax
from jax.experimental import pallas as pl
from jax.experimental.pallas import tpu as pltpu
from jax.experimental.pallas import tpu_sc as plsc

DIM = 64
PDIM = 32
BLK = 128
LANES = 16
NUM_WORKERS = 32
CHUNK = 28
BPW = 196
MM_BLK = 1000


def _mm_kernel(x_ref, w_ref, o_ref):
    o_ref[...] = jnp.dot(x_ref[...], w_ref[...],
                         preferred_element_type=jnp.float32)


def _transform(emb, w_r):
    n, d = emb.shape
    return pl.pallas_call(
        _mm_kernel,
        grid=(n // MM_BLK,),
        in_specs=[
            pl.BlockSpec((MM_BLK, d), lambda i: (i, 0)),
            pl.BlockSpec((d, d), lambda i: (0, 0)),
        ],
        out_specs=pl.BlockSpec((MM_BLK, d), lambda i: (i, 0)),
        out_shape=jax.ShapeDtypeStruct((n, d), jnp.float32),
    )(emb, w_r)


def _block_dot(ra_v, rb_v, o_v, orow):
    @pl.loop(0, BLK, step=LANES)
    def _(base):
        lane = lax.broadcasted_iota(jnp.int32, (LANES,), 0)
        rows = base + lane
        accs = [jnp.zeros((LANES,), jnp.float32) for _ in range(8)]
        for dcol in range(PDIM):
            cols = (lane + dcol) & (PDIM - 1)
            apk = plsc.load_gather(ra_v, [rows, cols])
            bpk = plsc.load_gather(rb_v, [rows, cols])
            ae, ao = plsc.unpack(plsc.bitcast(apk, jnp.bfloat16),
                                 format=plsc.PackFormat.INTERLEAVED,
                                 preferred_element_type=jnp.float32)
            be, bo = plsc.unpack(plsc.bitcast(bpk, jnp.bfloat16),
                                 format=plsc.PackFormat.INTERLEAVED,
                                 preferred_element_type=jnp.float32)
            accs[2 * (dcol % 4)] = accs[2 * (dcol % 4)] + ae * be
            accs[2 * (dcol % 4) + 1] = accs[2 * (dcol % 4) + 1] + ao * bo
        acc01 = (accs[0] + accs[1]) + (accs[2] + accs[3])
        acc23 = (accs[4] + accs[5]) + (accs[6] + accs[7])
        o_v[orow, pl.ds(base, LANES)] = acc01 + acc23


def _sc_scores(tab_left, tab_right, left2d, right2d):
    nb = left2d.shape[0]
    mesh = plsc.VectorSubcoreMesh(core_axis_name="c", subcore_axis_name="s")
    cp = pltpu.CompilerParams()
    for fld, val in (("needs_layout_passes", False),
                     ("use_tc_tiling_on_sc", False)):
        if fld in pltpu.CompilerParams.__dataclass_fields__:
            cp = dataclasses.replace(cp, **{fld: val})

    @functools.partial(
        pl.kernel,
        mesh=mesh,
        compiler_params=cp,
        out_type=jax.ShapeDtypeStruct((nb, BLK), jnp.float32),
        scratch_types=[
            pltpu.VMEM((CHUNK, BLK), jnp.int32),
            pltpu.VMEM((CHUNK, BLK), jnp.int32),
            pltpu.VMEM((BLK, PDIM), jnp.int32),
            pltpu.VMEM((BLK, PDIM), jnp.int32),
            pltpu.VMEM((BLK, PDIM), jnp.int32),
            pltpu.VMEM((BLK, PDIM), jnp.int32),
            pltpu.VMEM((CHUNK, BLK), jnp.float32),
            pltpu.SemaphoreType.DMA,
            pltpu.SemaphoreType.DMA,
            pltpu.SemaphoreType.DMA,
            pltpu.SemaphoreType.DMA,
        ],
    )
    def k(tl_hbm, tr_hbm, li_hbm, ri_hbm, o_hbm,
          li_v, ri_v, ra0_v, ra1_v, rb0_v, rb1_v, o_v,
          sa0, sa1, sb0, sb1):
        wid = lax.axis_index("s") * 2 + lax.axis_index("c")
        lo = wid * BPW

        @pl.loop(0, BPW // CHUNK)
        def _(c):
            start = lo + c * CHUNK
            pltpu.sync_copy(li_hbm.at[pl.ds(start, CHUNK)], li_v)
            pltpu.sync_copy(ri_hbm.at[pl.ds(start, CHUNK)], ri_v)

            pltpu.async_copy(tl_hbm.at[li_v.at[0]], ra0_v, sa0)
            pltpu.async_copy(tr_hbm.at[ri_v.at[0]], rb0_v, sb0)

            @pl.loop(0, CHUNK, step=2)
            def _(j):
                pltpu.async_copy(tl_hbm.at[li_v.at[j + 1]], ra1_v, sa1)
                pltpu.async_copy(tr_hbm.at[ri_v.at[j + 1]], rb1_v, sb1)
                pltpu.make_async_copy(tl_hbm.at[li_v.at[j]], ra0_v, sa0).wait()
                pltpu.make_async_copy(tr_hbm.at[ri_v.at[j]], rb0_v, sb0).wait()
                _block_dot(ra0_v, rb0_v, o_v, j)

                @pl.when(j + 2 < CHUNK)
                def _():
                    pltpu.async_copy(tl_hbm.at[li_v.at[j + 2]], ra0_v, sa0)
                    pltpu.async_copy(tr_hbm.at[ri_v.at[j + 2]], rb0_v, sb0)

                pltpu.make_async_copy(
                    tl_hbm.at[li_v.at[j + 1]], ra1_v, sa1).wait()
                pltpu.make_async_copy(
                    tr_hbm.at[ri_v.at[j + 1]], rb1_v, sb1).wait()
                _block_dot(ra1_v, rb1_v, o_v, j + 1)

            pltpu.sync_copy(o_v, o_hbm.at[pl.ds(start, CHUNK)])

    return k(tab_left, tab_right, left2d, right2d)


def kernel(embeddings, edge_index, W, r_id):
    n, d = embeddings.shape
    num_e = edge_index.shape[1]
    w_r = jnp.asarray(W)[r_id]

    def _pack(tab):
        tb = tab.astype(jnp.bfloat16).reshape(n, PDIM, 2)
        return jax.lax.bitcast_convert_type(tb, jnp.int32)

    tab_left = _pack(_transform(embeddings, w_r))
    tab_right = _pack(embeddings)
    nb = num_e // BLK
    nb_pad = NUM_WORKERS * BPW
    left = edge_index[0].reshape(nb, BLK)
    right = edge_index[1].reshape(nb, BLK)
    pad = ((0, nb_pad - nb), (0, 0))
    left = jnp.pad(left, pad)
    right = jnp.pad(right, pad)
    out = _sc_scores(tab_left, tab_right, left, right)
    return out[:nb].reshape(num_e)


# speedup vs baseline: 1.0239x; 1.0239x over previous
import dataclasses
import functools

import jax
import jax.numpy as jnp
from jax import lax
from jax.experimental import pallas as pl
from jax.experimental.pallas import tpu as pltpu
from jax.experimental.pallas import tpu_sc as plsc

DIM = 64
BLK = 128
LANES = 16
NUM_WORKERS = 32
CHUNK = 28
BPW = 196
MM_BLK = 1000


def _mm_kernel(x_ref, w_ref, o_ref):
    o_ref[...] = jnp.dot(x_ref[...], w_ref[...],
                         preferred_element_type=jnp.float32)


def _transform(emb, w_r):
    n, d = emb.shape
    return pl.pallas_call(
        _mm_kernel,
        grid=(n // MM_BLK,),
        in_specs=[
            pl.BlockSpec((MM_BLK, d), lambda i: (i, 0)),
            pl.BlockSpec((d, d), lambda i: (0, 0)),
        ],
        out_specs=pl.BlockSpec((MM_BLK, d), lambda i: (i, 0)),
        out_shape=jax.ShapeDtypeStruct((n, d), jnp.float32),
    )(emb, w_r)


def _block_dot(ra_v, rb_v, o_v, orow):
    @pl.loop(0, BLK, step=LANES)
    def _(base):
        lane = lax.broadcasted_iota(jnp.int32, (LANES,), 0)
        rows = base + lane
        accs = [jnp.zeros((LANES,), jnp.float32) for _ in range(4)]
        for dcol in range(DIM):
            cols = (lane + dcol) & (DIM - 1)
            a = plsc.load_gather(ra_v, [rows, cols])
            b = plsc.load_gather(rb_v, [rows, cols])
            accs[dcol % 4] = accs[dcol % 4] + a * b
        acc = (accs[0] + accs[1]) + (accs[2] + accs[3])
        o_v[orow, pl.ds(base, LANES)] = acc


def _sc_scores(tab_left, tab_right, left2d, right2d):
    nb = left2d.shape[0]
    mesh = plsc.VectorSubcoreMesh(core_axis_name="c", subcore_axis_name="s")
    cp = pltpu.CompilerParams()
    for fld, val in (("needs_layout_passes", False),
                     ("use_tc_tiling_on_sc", False)):
        if fld in pltpu.CompilerParams.__dataclass_fields__:
            cp = dataclasses.replace(cp, **{fld: val})

    @functools.partial(
        pl.kernel,
        mesh=mesh,
        compiler_params=cp,
        out_type=jax.ShapeDtypeStruct((nb, BLK), jnp.float32),
        scratch_types=[
            pltpu.VMEM((CHUNK, BLK), jnp.int32),
            pltpu.VMEM((CHUNK, BLK), jnp.int32),
            pltpu.VMEM((BLK, DIM), jnp.float32),
            pltpu.VMEM((BLK, DIM), jnp.float32),
            pltpu.VMEM((BLK, DIM), jnp.float32),
            pltpu.VMEM((BLK, DIM), jnp.float32),
            pltpu.VMEM((CHUNK, BLK), jnp.float32),
            pltpu.SemaphoreType.DMA,
            pltpu.SemaphoreType.DMA,
            pltpu.SemaphoreType.DMA,
            pltpu.SemaphoreType.DMA,
        ],
    )
    def k(tl_hbm, tr_hbm, li_hbm, ri_hbm, o_hbm,
          li_v, ri_v, ra0_v, ra1_v, rb0_v, rb1_v, o_v,
          sa0, sa1, sb0, sb1):
        wid = lax.axis_index("s") * 2 + lax.axis_index("c")
        lo = jnp.minimum(wid * BPW, nb - BPW)

        @pl.loop(0, BPW // CHUNK)
        def _(c):
            start = lo + c * CHUNK
            pltpu.sync_copy(li_hbm.at[pl.ds(start, CHUNK)], li_v)
            pltpu.sync_copy(ri_hbm.at[pl.ds(start, CHUNK)], ri_v)

            pltpu.async_copy(tl_hbm.at[li_v.at[0]], ra0_v, sa0)
            pltpu.async_copy(tr_hbm.at[ri_v.at[0]], rb0_v, sb0)

            @pl.loop(0, CHUNK, step=2)
            def _(j):
                pltpu.async_copy(tl_hbm.at[li_v.at[j + 1]], ra1_v, sa1)
                pltpu.async_copy(tr_hbm.at[ri_v.at[j + 1]], rb1_v, sb1)
                pltpu.make_async_copy(tl_hbm.at[li_v.at[j]], ra0_v, sa0).wait()
                pltpu.make_async_copy(tr_hbm.at[ri_v.at[j]], rb0_v, sb0).wait()
                _block_dot(ra0_v, rb0_v, o_v, j)

                @pl.when(j + 2 < CHUNK)
                def _():
                    pltpu.async_copy(tl_hbm.at[li_v.at[j + 2]], ra0_v, sa0)
                    pltpu.async_copy(tr_hbm.at[ri_v.at[j + 2]], rb0_v, sb0)

                pltpu.make_async_copy(
                    tl_hbm.at[li_v.at[j + 1]], ra1_v, sa1).wait()
                pltpu.make_async_copy(
                    tr_hbm.at[ri_v.at[j + 1]], rb1_v, sb1).wait()
                _block_dot(ra1_v, rb1_v, o_v, j + 1)

            pltpu.sync_copy(o_v, o_hbm.at[pl.ds(start, CHUNK)])

    return k(tab_left, tab_right, left2d, right2d)


def kernel(embeddings, edge_index, W, r_id):
    n, d = embeddings.shape
    num_e = edge_index.shape[1]
    w_r = jnp.asarray(W)[r_id]
    tab_left = _transform(embeddings, w_r)
    nb = num_e // BLK
    left = edge_index[0].reshape(nb, BLK)
    right = edge_index[1].reshape(nb, BLK)
    out = _sc_scores(tab_left, embeddings, left, right)
    return out.reshape(num_e)


# speedup vs baseline: 1.7242x; 1.6840x over previous
import dataclasses
import functools

import jax
import jax.numpy as jnp
from jax import lax
from jax.experimental import pallas as pl
from jax.experimental.pallas import tpu as pltpu
from jax.experimental.pallas import tpu_sc as plsc

DIM = 64
PDIM = 32
BLK = 128
LANES = 16
NUM_WORKERS = 32
CHUNK = 28
BPW = 196
MM_BLK = 1000


def _pack_bf16(x):
    lo = lax.bitcast_convert_type(
        x[:, :PDIM].astype(jnp.bfloat16), jnp.uint16).astype(jnp.uint32)
    hi = lax.bitcast_convert_type(
        x[:, PDIM:].astype(jnp.bfloat16), jnp.uint16).astype(jnp.uint32)
    return lax.bitcast_convert_type(lo | (hi << 16), jnp.int32)


def _mm_kernel(x_ref, w_ref, t_ref, e_ref):
    x = x_ref[...]
    t_ref[...] = _pack_bf16(jnp.dot(x, w_ref[...],
                                    preferred_element_type=jnp.float32))
    e_ref[...] = _pack_bf16(x)


def _transform(emb, w_r):
    n, d = emb.shape
    return pl.pallas_call(
        _mm_kernel,
        grid=(n // MM_BLK,),
        in_specs=[
            pl.BlockSpec((MM_BLK, d), lambda i: (i, 0)),
            pl.BlockSpec((d, d), lambda i: (0, 0)),
        ],
        out_specs=[
            pl.BlockSpec((MM_BLK, PDIM), lambda i: (i, 0)),
            pl.BlockSpec((MM_BLK, PDIM), lambda i: (i, 0)),
        ],
        out_shape=[
            jax.ShapeDtypeStruct((n, PDIM), jnp.int32),
            jax.ShapeDtypeStruct((n, PDIM), jnp.int32),
        ],
    )(emb, w_r)


def _block_dot(ra_v, rb_v, o_v, orow):
    @pl.loop(0, BLK, step=LANES)
    def _(base):
        lane = lax.broadcasted_iota(jnp.int32, (LANES,), 0)
        rows = base + lane
        accs = [jnp.zeros((LANES,), jnp.float32) for _ in range(8)]
        for dcol in range(PDIM):
            cols = (lane + dcol) & (PDIM - 1)
            apk = plsc.load_gather(ra_v, [rows, cols])
            bpk = plsc.load_gather(rb_v, [rows, cols])
            ae, ao = plsc.unpack(plsc.bitcast(apk, jnp.bfloat16),
                                 format=plsc.PackFormat.INTERLEAVED,
                                 preferred_element_type=jnp.float32)
            be, bo = plsc.unpack(plsc.bitcast(bpk, jnp.bfloat16),
                                 format=plsc.PackFormat.INTERLEAVED,
                                 preferred_element_type=jnp.float32)
            accs[2 * (dcol % 4)] = accs[2 * (dcol % 4)] + ae * be
            accs[2 * (dcol % 4) + 1] = accs[2 * (dcol % 4) + 1] + ao * bo
        acc01 = (accs[0] + accs[1]) + (accs[2] + accs[3])
        acc23 = (accs[4] + accs[5]) + (accs[6] + accs[7])
        o_v[orow, pl.ds(base, LANES)] = acc01 + acc23


def _sc_scores(tab_left, tab_right, left2d, right2d):
    nb = left2d.shape[0]
    mesh = plsc.VectorSubcoreMesh(core_axis_name="c", subcore_axis_name="s")
    cp = pltpu.CompilerParams()
    for fld, val in (("needs_layout_passes", False),
                     ("use_tc_tiling_on_sc", False)):
        if fld in pltpu.CompilerParams.__dataclass_fields__:
            cp = dataclasses.replace(cp, **{fld: val})

    @functools.partial(
        pl.kernel,
        mesh=mesh,
        compiler_params=cp,
        out_type=jax.ShapeDtypeStruct((nb, BLK), jnp.float32),
        scratch_types=[
            pltpu.VMEM((CHUNK, BLK), jnp.int32),
            pltpu.VMEM((CHUNK, BLK), jnp.int32),
            pltpu.VMEM((BLK, PDIM), jnp.int32),
            pltpu.VMEM((BLK, PDIM), jnp.int32),
            pltpu.VMEM((BLK, PDIM), jnp.int32),
            pltpu.VMEM((BLK, PDIM), jnp.int32),
            pltpu.VMEM((CHUNK, BLK), jnp.float32),
            pltpu.SemaphoreType.DMA,
            pltpu.SemaphoreType.DMA,
            pltpu.SemaphoreType.DMA,
            pltpu.SemaphoreType.DMA,
        ],
    )
    def k(tl_hbm, tr_hbm, li_hbm, ri_hbm, o_hbm,
          li_v, ri_v, ra0_v, ra1_v, rb0_v, rb1_v, o_v,
          sa0, sa1, sb0, sb1):
        wid = lax.axis_index("s") * 2 + lax.axis_index("c")
        lo = jnp.minimum(wid * BPW, nb - BPW)

        @pl.loop(0, BPW // CHUNK)
        def _(c):
            start = lo + c * CHUNK
            pltpu.sync_copy(li_hbm.at[pl.ds(start, CHUNK)], li_v)
            pltpu.sync_copy(ri_hbm.at[pl.ds(start, CHUNK)], ri_v)

            pltpu.async_copy(tl_hbm.at[li_v.at[0]], ra0_v, sa0)
            pltpu.async_copy(tr_hbm.at[ri_v.at[0]], rb0_v, sb0)

            @pl.loop(0, CHUNK, step=2)
            def _(j):
                pltpu.async_copy(tl_hbm.at[li_v.at[j + 1]], ra1_v, sa1)
                pltpu.async_copy(tr_hbm.at[ri_v.at[j + 1]], rb1_v, sb1)
                pltpu.make_async_copy(tl_hbm.at[li_v.at[j]], ra0_v, sa0).wait()
                pltpu.make_async_copy(tr_hbm.at[ri_v.at[j]], rb0_v, sb0).wait()
                _block_dot(ra0_v, rb0_v, o_v, j)

                @pl.when(j + 2 < CHUNK)
                def _():
                    pltpu.async_copy(tl_hbm.at[li_v.at[j + 2]], ra0_v, sa0)
                    pltpu.async_copy(tr_hbm.at[ri_v.at[j + 2]], rb0_v, sb0)

                pltpu.make_async_copy(
                    tl_hbm.at[li_v.at[j + 1]], ra1_v, sa1).wait()
                pltpu.make_async_copy(
                    tr_hbm.at[ri_v.at[j + 1]], rb1_v, sb1).wait()
                _block_dot(ra1_v, rb1_v, o_v, j + 1)

            pltpu.sync_copy(o_v, o_hbm.at[pl.ds(start, CHUNK)])

    return k(tab_left, tab_right, left2d, right2d)


def kernel(embeddings, edge_index, W, r_id):
    n, d = embeddings.shape
    num_e = edge_index.shape[1]
    w_r = jnp.asarray(W)[r_id]
    tab_left, tab_right = _transform(embeddings, w_r)
    nb = num_e // BLK
    left = edge_index[0].reshape(nb, BLK)
    right = edge_index[1].reshape(nb, BLK)
    out = _sc_scores(tab_left, tab_right, left, right)
    return out.reshape(num_e)


# speedup vs baseline: 1.9600x; 1.1368x over previous
import dataclasses
import functools

import jax
import jax.numpy as jnp
from jax import lax
from jax.experimental import pallas as pl
from jax.experimental.pallas import tpu as pltpu
from jax.experimental.pallas import tpu_sc as plsc

DIM = 64
PDIM = 32
BLK = 128
LANES = 16
NUM_WORKERS = 32
CHUNK = 98
BPW = 196
MM_BLK = 1000


def _pack_bf16(x):
    lo = lax.bitcast_convert_type(
        x[:, :PDIM].astype(jnp.bfloat16), jnp.uint16).astype(jnp.uint32)
    hi = lax.bitcast_convert_type(
        x[:, PDIM:].astype(jnp.bfloat16), jnp.uint16).astype(jnp.uint32)
    return lax.bitcast_convert_type(lo | (hi << 16), jnp.int32)


def _mm_kernel(x_ref, w_ref, t_ref, e_ref):
    x = x_ref[...]
    t_ref[...] = _pack_bf16(jnp.dot(x, w_ref[...],
                                    preferred_element_type=jnp.float32))
    e_ref[...] = _pack_bf16(x)


def _transform(emb, w_r):
    n, d = emb.shape
    return pl.pallas_call(
        _mm_kernel,
        grid=(n // MM_BLK,),
        in_specs=[
            pl.BlockSpec((MM_BLK, d), lambda i: (i, 0)),
            pl.BlockSpec((d, d), lambda i: (0, 0)),
        ],
        out_specs=[
            pl.BlockSpec((MM_BLK, PDIM), lambda i: (i, 0)),
            pl.BlockSpec((MM_BLK, PDIM), lambda i: (i, 0)),
        ],
        out_shape=[
            jax.ShapeDtypeStruct((n, PDIM), jnp.int32),
            jax.ShapeDtypeStruct((n, PDIM), jnp.int32),
        ],
    )(emb, w_r)


def _block_dot(ra_v, rb_v, o_v, orow):
    @pl.loop(0, BLK, step=LANES)
    def _(base):
        lane = lax.broadcasted_iota(jnp.int32, (LANES,), 0)
        rows = base + lane
        accs = [jnp.zeros((LANES,), jnp.float32) for _ in range(8)]
        for dcol in range(PDIM):
            cols = (lane + dcol) & (PDIM - 1)
            apk = plsc.load_gather(ra_v, [rows, cols])
            bpk = plsc.load_gather(rb_v, [rows, cols])
            pab = plsc.bitcast(apk, jnp.bfloat16) * plsc.bitcast(bpk, jnp.bfloat16)
            pe, po = plsc.unpack(pab, format=plsc.PackFormat.INTERLEAVED,
                                 preferred_element_type=jnp.float32)
            accs[2 * (dcol % 4)] = accs[2 * (dcol % 4)] + pe
            accs[2 * (dcol % 4) + 1] = accs[2 * (dcol % 4) + 1] + po
        acc01 = (accs[0] + accs[1]) + (accs[2] + accs[3])
        acc23 = (accs[4] + accs[5]) + (accs[6] + accs[7])
        o_v[orow, pl.ds(base, LANES)] = acc01 + acc23


def _sc_scores(tab_left, tab_right, left2d, right2d):
    nb = left2d.shape[0]
    mesh = plsc.VectorSubcoreMesh(core_axis_name="c", subcore_axis_name="s")
    cp = pltpu.CompilerParams()
    for fld, val in (("needs_layout_passes", False),
                     ("use_tc_tiling_on_sc", False)):
        if fld in pltpu.CompilerParams.__dataclass_fields__:
            cp = dataclasses.replace(cp, **{fld: val})

    @functools.partial(
        pl.kernel,
        mesh=mesh,
        compiler_params=cp,
        out_type=jax.ShapeDtypeStruct((nb, BLK), jnp.float32),
        scratch_types=[
            pltpu.VMEM((CHUNK, BLK), jnp.int32),
            pltpu.VMEM((CHUNK, BLK), jnp.int32),
            pltpu.VMEM((BLK, PDIM), jnp.int32),
            pltpu.VMEM((BLK, PDIM), jnp.int32),
            pltpu.VMEM((BLK, PDIM), jnp.int32),
            pltpu.VMEM((BLK, PDIM), jnp.int32),
            pltpu.VMEM((CHUNK, BLK), jnp.float32),
            pltpu.SemaphoreType.DMA,
            pltpu.SemaphoreType.DMA,
            pltpu.SemaphoreType.DMA,
            pltpu.SemaphoreType.DMA,
        ],
    )
    def k(tl_hbm, tr_hbm, li_hbm, ri_hbm, o_hbm,
          li_v, ri_v, ra0_v, ra1_v, rb0_v, rb1_v, o_v,
          sa0, sa1, sb0, sb1):
        wid = lax.axis_index("s") * 2 + lax.axis_index("c")
        lo = jnp.minimum(wid * BPW, nb - BPW)

        @pl.loop(0, BPW // CHUNK)
        def _(c):
            start = lo + c * CHUNK
            pltpu.sync_copy(li_hbm.at[pl.ds(start, CHUNK)], li_v)
            pltpu.sync_copy(ri_hbm.at[pl.ds(start, CHUNK)], ri_v)

            pltpu.async_copy(tl_hbm.at[li_v.at[0]], ra0_v, sa0)
            pltpu.async_copy(tr_hbm.at[ri_v.at[0]], rb0_v, sb0)

            @pl.loop(0, CHUNK, step=2)
            def _(j):
                pltpu.async_copy(tl_hbm.at[li_v.at[j + 1]], ra1_v, sa1)
                pltpu.async_copy(tr_hbm.at[ri_v.at[j + 1]], rb1_v, sb1)
                pltpu.make_async_copy(tl_hbm.at[li_v.at[j]], ra0_v, sa0).wait()
                pltpu.make_async_copy(tr_hbm.at[ri_v.at[j]], rb0_v, sb0).wait()
                _block_dot(ra0_v, rb0_v, o_v, j)

                @pl.when(j + 2 < CHUNK)
                def _():
                    pltpu.async_copy(tl_hbm.at[li_v.at[j + 2]], ra0_v, sa0)
                    pltpu.async_copy(tr_hbm.at[ri_v.at[j + 2]], rb0_v, sb0)

                pltpu.make_async_copy(
                    tl_hbm.at[li_v.at[j + 1]], ra1_v, sa1).wait()
                pltpu.make_async_copy(
                    tr_hbm.at[ri_v.at[j + 1]], rb1_v, sb1).wait()
                _block_dot(ra1_v, rb1_v, o_v, j + 1)

            pltpu.sync_copy(o_v, o_hbm.at[pl.ds(start, CHUNK)])

    return k(tab_left, tab_right, left2d, right2d)


def kernel(embeddings, edge_index, W, r_id):
    n, d = embeddings.shape
    num_e = edge_index.shape[1]
    w_r = jnp.asarray(W)[r_id]
    tab_left, tab_right = _transform(embeddings, w_r)
    nb = num_e // BLK
    left = edge_index[0].reshape(nb, BLK)
    right = edge_index[1].reshape(nb, BLK)
    out = _sc_scores(tab_left, tab_right, left, right)
    return out.reshape(num_e)


# speedup vs baseline: 2.1277x; 1.0855x over previous
import dataclasses
import functools

import jax
import jax.numpy as jnp
from jax import lax
from jax.experimental import pallas as pl
from jax.experimental.pallas import tpu as pltpu
from jax.experimental.pallas import tpu_sc as plsc

DIM = 64
PDIM = 32
BLK = 128
LANES = 16
NUM_WORKERS = 32
CHUNK = 98
BPW = 196
MM_BLK = 10000


def _pack_bf16(x):
    lo = lax.bitcast_convert_type(
        x[:, :PDIM].astype(jnp.bfloat16), jnp.uint16).astype(jnp.uint32)
    hi = lax.bitcast_convert_type(
        x[:, PDIM:].astype(jnp.bfloat16), jnp.uint16).astype(jnp.uint32)
    return lax.bitcast_convert_type(lo | (hi << 16), jnp.int32)


def _mm_kernel(x_ref, w_ref, t_ref, e_ref):
    x = x_ref[...]
    t_ref[...] = _pack_bf16(jnp.dot(x, w_ref[...],
                                    preferred_element_type=jnp.float32))
    e_ref[...] = _pack_bf16(x)


def _transform(emb, w_r):
    n, d = emb.shape
    return pl.pallas_call(
        _mm_kernel,
        grid=(n // MM_BLK,),
        in_specs=[
            pl.BlockSpec((MM_BLK, d), lambda i: (i, 0)),
            pl.BlockSpec((d, d), lambda i: (0, 0)),
        ],
        out_specs=[
            pl.BlockSpec((MM_BLK, PDIM), lambda i: (i, 0)),
            pl.BlockSpec((MM_BLK, PDIM), lambda i: (i, 0)),
        ],
        out_shape=[
            jax.ShapeDtypeStruct((n, PDIM), jnp.int32),
            jax.ShapeDtypeStruct((n, PDIM), jnp.int32),
        ],
    )(emb, w_r)


def _block_dot(ra_v, rb_v, o_v, orow):
    @pl.loop(0, BLK, step=LANES)
    def _(base):
        lane = lax.broadcasted_iota(jnp.int32, (LANES,), 0)
        rows = base + lane
        accs = [jnp.zeros((LANES,), jnp.float32) for _ in range(8)]
        for dcol in range(PDIM):
            cols = (lane + dcol) & (PDIM - 1)
            apk = plsc.load_gather(ra_v, [rows, cols])
            bpk = plsc.load_gather(rb_v, [rows, cols])
            pab = plsc.bitcast(apk, jnp.bfloat16) * plsc.bitcast(bpk, jnp.bfloat16)
            pe, po = plsc.unpack(pab, format=plsc.PackFormat.INTERLEAVED,
                                 preferred_element_type=jnp.float32)
            accs[2 * (dcol % 4)] = accs[2 * (dcol % 4)] + pe
            accs[2 * (dcol % 4) + 1] = accs[2 * (dcol % 4) + 1] + po
        acc01 = (accs[0] + accs[1]) + (accs[2] + accs[3])
        acc23 = (accs[4] + accs[5]) + (accs[6] + accs[7])
        o_v[orow, pl.ds(base, LANES)] = acc01 + acc23


def _sc_scores(tab_left, tab_right, left2d, right2d):
    nb = left2d.shape[0]
    mesh = plsc.VectorSubcoreMesh(core_axis_name="c", subcore_axis_name="s")
    cp = pltpu.CompilerParams()
    for fld, val in (("needs_layout_passes", False),
                     ("use_tc_tiling_on_sc", False)):
        if fld in pltpu.CompilerParams.__dataclass_fields__:
            cp = dataclasses.replace(cp, **{fld: val})

    @functools.partial(
        pl.kernel,
        mesh=mesh,
        compiler_params=cp,
        out_type=jax.ShapeDtypeStruct((nb, BLK), jnp.float32),
        scratch_types=[
            pltpu.VMEM((CHUNK, BLK), jnp.int32),
            pltpu.VMEM((CHUNK, BLK), jnp.int32),
            pltpu.VMEM((BLK, PDIM), jnp.int32),
            pltpu.VMEM((BLK, PDIM), jnp.int32),
            pltpu.VMEM((BLK, PDIM), jnp.int32),
            pltpu.VMEM((BLK, PDIM), jnp.int32),
            pltpu.VMEM((CHUNK, BLK), jnp.float32),
            pltpu.SemaphoreType.DMA,
            pltpu.SemaphoreType.DMA,
            pltpu.SemaphoreType.DMA,
            pltpu.SemaphoreType.DMA,
        ],
    )
    def k(tl_hbm, tr_hbm, li_hbm, ri_hbm, o_hbm,
          li_v, ri_v, ra0_v, ra1_v, rb0_v, rb1_v, o_v,
          sa0, sa1, sb0, sb1):
        wid = lax.axis_index("s") * 2 + lax.axis_index("c")
        lo = jnp.minimum(wid * BPW, nb - BPW)

        @pl.loop(0, BPW // CHUNK)
        def _(c):
            start = lo + c * CHUNK
            pltpu.sync_copy(li_hbm.at[pl.ds(start, CHUNK)], li_v)
            pltpu.sync_copy(ri_hbm.at[pl.ds(start, CHUNK)], ri_v)

            pltpu.async_copy(tl_hbm.at[li_v.at[0]], ra0_v, sa0)
            pltpu.async_copy(tr_hbm.at[ri_v.at[0]], rb0_v, sb0)

            @pl.loop(0, CHUNK, step=2)
            def _(j):
                pltpu.async_copy(tl_hbm.at[li_v.at[j + 1]], ra1_v, sa1)
                pltpu.async_copy(tr_hbm.at[ri_v.at[j + 1]], rb1_v, sb1)
                pltpu.make_async_copy(tl_hbm.at[li_v.at[j]], ra0_v, sa0).wait()
                pltpu.make_async_copy(tr_hbm.at[ri_v.at[j]], rb0_v, sb0).wait()
                _block_dot(ra0_v, rb0_v, o_v, j)

                @pl.when(j + 2 < CHUNK)
                def _():
                    pltpu.async_copy(tl_hbm.at[li_v.at[j + 2]], ra0_v, sa0)
                    pltpu.async_copy(tr_hbm.at[ri_v.at[j + 2]], rb0_v, sb0)

                pltpu.make_async_copy(
                    tl_hbm.at[li_v.at[j + 1]], ra1_v, sa1).wait()
                pltpu.make_async_copy(
                    tr_hbm.at[ri_v.at[j + 1]], rb1_v, sb1).wait()
                _block_dot(ra1_v, rb1_v, o_v, j + 1)

            pltpu.sync_copy(o_v, o_hbm.at[pl.ds(start, CHUNK)])

    return k(tab_left, tab_right, left2d, right2d)


def kernel(embeddings, edge_index, W, r_id):
    n, d = embeddings.shape
    num_e = edge_index.shape[1]
    w_r = jnp.asarray(W)[r_id]
    tab_left, tab_right = _transform(embeddings, w_r)
    nb = num_e // BLK
    left = edge_index[0].reshape(nb, BLK)
    right = edge_index[1].reshape(nb, BLK)
    out = _sc_scores(tab_left, tab_right, left, right)
    return out.reshape(num_e)


# speedup vs baseline: 2.1412x; 1.0064x over previous
import dataclasses
import functools

import jax
import jax.numpy as jnp
from jax import lax
from jax.experimental import pallas as pl
from jax.experimental.pallas import tpu as pltpu
from jax.experimental.pallas import tpu_sc as plsc

DIM = 64
PDIM = 32
BLK = 128
LANES = 16
NUM_WORKERS = 32
CHUNK = 196
BPW = 196
MM_BLK = 10000


def _pack_bf16(x):
    lo = lax.bitcast_convert_type(
        x[:, :PDIM].astype(jnp.bfloat16), jnp.uint16).astype(jnp.uint32)
    hi = lax.bitcast_convert_type(
        x[:, PDIM:].astype(jnp.bfloat16), jnp.uint16).astype(jnp.uint32)
    return lax.bitcast_convert_type(lo | (hi << 16), jnp.int32)


def _mm_kernel(x_ref, w_ref, t_ref, e_ref):
    x = x_ref[...]
    t_ref[...] = _pack_bf16(jnp.dot(x, w_ref[...],
                                    preferred_element_type=jnp.float32))
    e_ref[...] = _pack_bf16(x)


def _transform(emb, w_r):
    n, d = emb.shape
    return pl.pallas_call(
        _mm_kernel,
        grid=(n // MM_BLK,),
        in_specs=[
            pl.BlockSpec((MM_BLK, d), lambda i: (i, 0)),
            pl.BlockSpec((d, d), lambda i: (0, 0)),
        ],
        out_specs=[
            pl.BlockSpec((MM_BLK, PDIM), lambda i: (i, 0)),
            pl.BlockSpec((MM_BLK, PDIM), lambda i: (i, 0)),
        ],
        out_shape=[
            jax.ShapeDtypeStruct((n, PDIM), jnp.int32),
            jax.ShapeDtypeStruct((n, PDIM), jnp.int32),
        ],
    )(emb, w_r)


def _block_dot(ra_v, rb_v, o_v, orow):
    @pl.loop(0, BLK, step=LANES)
    def _(base):
        lane = lax.broadcasted_iota(jnp.int32, (LANES,), 0)
        rows = base + lane
        accs = [jnp.zeros((LANES,), jnp.float32) for _ in range(8)]
        for dcol in range(PDIM):
            cols = (lane + dcol) & (PDIM - 1)
            apk = plsc.load_gather(ra_v, [rows, cols])
            bpk = plsc.load_gather(rb_v, [rows, cols])
            pab = plsc.bitcast(apk, jnp.bfloat16) * plsc.bitcast(bpk, jnp.bfloat16)
            pe, po = plsc.unpack(pab, format=plsc.PackFormat.INTERLEAVED,
                                 preferred_element_type=jnp.float32)
            accs[2 * (dcol % 4)] = accs[2 * (dcol % 4)] + pe
            accs[2 * (dcol % 4) + 1] = accs[2 * (dcol % 4) + 1] + po
        acc01 = (accs[0] + accs[1]) + (accs[2] + accs[3])
        acc23 = (accs[4] + accs[5]) + (accs[6] + accs[7])
        o_v[orow, pl.ds(base, LANES)] = acc01 + acc23


def _sc_scores(tab_left, tab_right, left2d, right2d):
    nb = left2d.shape[0]
    mesh = plsc.VectorSubcoreMesh(core_axis_name="c", subcore_axis_name="s")
    cp = pltpu.CompilerParams()
    for fld, val in (("needs_layout_passes", False),
                     ("use_tc_tiling_on_sc", False)):
        if fld in pltpu.CompilerParams.__dataclass_fields__:
            cp = dataclasses.replace(cp, **{fld: val})

    @functools.partial(
        pl.kernel,
        mesh=mesh,
        compiler_params=cp,
        out_type=jax.ShapeDtypeStruct((nb, BLK), jnp.float32),
        scratch_types=[
            pltpu.VMEM((CHUNK, BLK), jnp.int32),
            pltpu.VMEM((CHUNK, BLK), jnp.int32),
            pltpu.VMEM((BLK, PDIM), jnp.int32),
            pltpu.VMEM((BLK, PDIM), jnp.int32),
            pltpu.VMEM((BLK, PDIM), jnp.int32),
            pltpu.VMEM((BLK, PDIM), jnp.int32),
            pltpu.VMEM((CHUNK, BLK), jnp.float32),
            pltpu.SemaphoreType.DMA,
            pltpu.SemaphoreType.DMA,
            pltpu.SemaphoreType.DMA,
            pltpu.SemaphoreType.DMA,
        ],
    )
    def k(tl_hbm, tr_hbm, li_hbm, ri_hbm, o_hbm,
          li_v, ri_v, ra0_v, ra1_v, rb0_v, rb1_v, o_v,
          sa0, sa1, sb0, sb1):
        wid = lax.axis_index("s") * 2 + lax.axis_index("c")
        lo = jnp.minimum(wid * BPW, nb - BPW)

        @pl.loop(0, BPW // CHUNK)
        def _(c):
            start = lo + c * CHUNK
            pltpu.sync_copy(li_hbm.at[pl.ds(start, CHUNK)], li_v)
            pltpu.sync_copy(ri_hbm.at[pl.ds(start, CHUNK)], ri_v)

            pltpu.async_copy(tl_hbm.at[li_v.at[0]], ra0_v, sa0)
            pltpu.async_copy(tr_hbm.at[ri_v.at[0]], rb0_v, sb0)

            @pl.loop(0, CHUNK, step=2)
            def _(j):
                pltpu.async_copy(tl_hbm.at[li_v.at[j + 1]], ra1_v, sa1)
                pltpu.async_copy(tr_hbm.at[ri_v.at[j + 1]], rb1_v, sb1)
                pltpu.make_async_copy(tl_hbm.at[li_v.at[j]], ra0_v, sa0).wait()
                pltpu.make_async_copy(tr_hbm.at[ri_v.at[j]], rb0_v, sb0).wait()
                _block_dot(ra0_v, rb0_v, o_v, j)

                @pl.when(j + 2 < CHUNK)
                def _():
                    pltpu.async_copy(tl_hbm.at[li_v.at[j + 2]], ra0_v, sa0)
                    pltpu.async_copy(tr_hbm.at[ri_v.at[j + 2]], rb0_v, sb0)

                pltpu.make_async_copy(
                    tl_hbm.at[li_v.at[j + 1]], ra1_v, sa1).wait()
                pltpu.make_async_copy(
                    tr_hbm.at[ri_v.at[j + 1]], rb1_v, sb1).wait()
                _block_dot(ra1_v, rb1_v, o_v, j + 1)

            pltpu.sync_copy(o_v, o_hbm.at[pl.ds(start, CHUNK)])

    return k(tab_left, tab_right, left2d, right2d)


def kernel(embeddings, edge_index, W, r_id):
    n, d = embeddings.shape
    num_e = edge_index.shape[1]
    w_r = jnp.asarray(W)[r_id]
    tab_left, tab_right = _transform(embeddings, w_r)
    nb = num_e // BLK
    left = edge_index[0].reshape(nb, BLK)
    right = edge_index[1].reshape(nb, BLK)
    out = _sc_scores(tab_left, tab_right, left, right)
    return out.reshape(num_e)


# speedup vs baseline: 2.4299x; 1.1348x over previous
import dataclasses
import functools

import jax
import jax.numpy as jnp
from jax import lax
from jax.experimental import pallas as pl
from jax.experimental.pallas import tpu as pltpu
from jax.experimental.pallas import tpu_sc as plsc

DIM = 64
PDIM = 32
BLK = 128
LANES = 16
NUM_WORKERS = 32
CHUNK = 196
BPW = 196
MM_BLK = 10000


def _pack_bf16(x):
    lo = lax.bitcast_convert_type(
        x[:, :PDIM].astype(jnp.bfloat16), jnp.uint16).astype(jnp.uint32)
    hi = lax.bitcast_convert_type(
        x[:, PDIM:].astype(jnp.bfloat16), jnp.uint16).astype(jnp.uint32)
    return lax.bitcast_convert_type(lo | (hi << 16), jnp.int32)


def _mm_kernel(x_ref, w_ref, t_ref, e_ref):
    x = x_ref[...]
    t_ref[...] = _pack_bf16(jnp.dot(x, w_ref[...],
                                    preferred_element_type=jnp.float32))
    e_ref[...] = _pack_bf16(x)


def _transform(emb, w_r):
    n, d = emb.shape
    return pl.pallas_call(
        _mm_kernel,
        grid=(n // MM_BLK,),
        in_specs=[
            pl.BlockSpec((MM_BLK, d), lambda i: (i, 0)),
            pl.BlockSpec((d, d), lambda i: (0, 0)),
        ],
        out_specs=[
            pl.BlockSpec((MM_BLK, PDIM), lambda i: (i, 0)),
            pl.BlockSpec((MM_BLK, PDIM), lambda i: (i, 0)),
        ],
        out_shape=[
            jax.ShapeDtypeStruct((n, PDIM), jnp.int32),
            jax.ShapeDtypeStruct((n, PDIM), jnp.int32),
        ],
    )(emb, w_r)


def _block_dot(ra_v, rb_v, o_v, orow):
    @pl.loop(0, BLK, step=LANES)
    def _(base):
        lane = lax.broadcasted_iota(jnp.int32, (LANES,), 0)
        rows = base + lane
        accs = [jnp.zeros((LANES,), jnp.float32) for _ in range(8)]
        for dcol in range(PDIM):
            cols = (lane + dcol) & (PDIM - 1)
            apk = plsc.load_gather(ra_v, [rows, cols])
            bpk = plsc.load_gather(rb_v, [rows, cols])
            pab = plsc.bitcast(apk, jnp.bfloat16) * plsc.bitcast(bpk, jnp.bfloat16)
            pe, po = plsc.unpack(pab, format=plsc.PackFormat.INTERLEAVED,
                                 preferred_element_type=jnp.float32)
            accs[2 * (dcol % 4)] = accs[2 * (dcol % 4)] + pe
            accs[2 * (dcol % 4) + 1] = accs[2 * (dcol % 4) + 1] + po
        acc01 = (accs[0] + accs[1]) + (accs[2] + accs[3])
        acc23 = (accs[4] + accs[5]) + (accs[6] + accs[7])
        o_v[orow, pl.ds(base, LANES)] = acc01 + acc23


def _sc_scores(tab_left, tab_right, left2d, right2d):
    nb = left2d.shape[0]
    mesh = plsc.VectorSubcoreMesh(core_axis_name="c", subcore_axis_name="s")
    cp = pltpu.CompilerParams()
    for fld, val in (("needs_layout_passes", False),
                     ("use_tc_tiling_on_sc", False)):
        if fld in pltpu.CompilerParams.__dataclass_fields__:
            cp = dataclasses.replace(cp, **{fld: val})

    @functools.partial(
        pl.kernel,
        mesh=mesh,
        compiler_params=cp,
        out_type=jax.ShapeDtypeStruct((nb, BLK), jnp.float32),
        scratch_types=[
            pltpu.VMEM((CHUNK, BLK), jnp.int32),
            pltpu.VMEM((CHUNK, BLK), jnp.int32),
            pltpu.VMEM((BLK, PDIM), jnp.int32),
            pltpu.VMEM((BLK, PDIM), jnp.int32),
            pltpu.VMEM((BLK, PDIM), jnp.int32),
            pltpu.VMEM((BLK, PDIM), jnp.int32),
            pltpu.VMEM((BLK, PDIM), jnp.int32),
            pltpu.VMEM((BLK, PDIM), jnp.int32),
            pltpu.VMEM((BLK, PDIM), jnp.int32),
            pltpu.VMEM((BLK, PDIM), jnp.int32),
            pltpu.VMEM((CHUNK, BLK), jnp.float32),
            pltpu.SemaphoreType.DMA,
            pltpu.SemaphoreType.DMA,
            pltpu.SemaphoreType.DMA,
            pltpu.SemaphoreType.DMA,
            pltpu.SemaphoreType.DMA,
            pltpu.SemaphoreType.DMA,
            pltpu.SemaphoreType.DMA,
            pltpu.SemaphoreType.DMA,
        ],
    )
    def k(tl_hbm, tr_hbm, li_hbm, ri_hbm, o_hbm,
          li_v, ri_v, ra0_v, ra1_v, ra2_v, ra3_v,
          rb0_v, rb1_v, rb2_v, rb3_v, o_v,
          sa0, sa1, sa2, sa3, sb0, sb1, sb2, sb3):
        ras = [ra0_v, ra1_v, ra2_v, ra3_v]
        rbs = [rb0_v, rb1_v, rb2_v, rb3_v]
        sas = [sa0, sa1, sa2, sa3]
        sbs = [sb0, sb1, sb2, sb3]
        wid = lax.axis_index("s") * 2 + lax.axis_index("c")
        lo = jnp.minimum(wid * BPW, nb - BPW)

        @pl.loop(0, BPW // CHUNK)
        def _(c):
            start = lo + c * CHUNK
            pltpu.sync_copy(li_hbm.at[pl.ds(start, CHUNK)], li_v)
            pltpu.sync_copy(ri_hbm.at[pl.ds(start, CHUNK)], ri_v)

            for p in range(3):
                pltpu.async_copy(tl_hbm.at[li_v.at[p]], ras[p], sas[p])
                pltpu.async_copy(tr_hbm.at[ri_v.at[p]], rbs[p], sbs[p])

            @pl.loop(0, CHUNK, step=4)
            def _(j):
                for p in range(4):
                    blk = j + p
                    nxt = (p + 3) % 4

                    @pl.when(blk + 3 < CHUNK)
                    def _():
                        pltpu.async_copy(
                            tl_hbm.at[li_v.at[blk + 3]], ras[nxt], sas[nxt])
                        pltpu.async_copy(
                            tr_hbm.at[ri_v.at[blk + 3]], rbs[nxt], sbs[nxt])

                    pltpu.make_async_copy(
                        tl_hbm.at[li_v.at[blk]], ras[p], sas[p]).wait()
                    pltpu.make_async_copy(
                        tr_hbm.at[ri_v.at[blk]], rbs[p], sbs[p]).wait()
                    _block_dot(ras[p], rbs[p], o_v, blk)

            pltpu.sync_copy(o_v, o_hbm.at[pl.ds(start, CHUNK)])

    return k(tab_left, tab_right, left2d, right2d)


def kernel(embeddings, edge_index, W, r_id):
    n, d = embeddings.shape
    num_e = edge_index.shape[1]
    w_r = jnp.asarray(W)[r_id]
    tab_left, tab_right = _transform(embeddings, w_r)
    nb = num_e // BLK
    left = edge_index[0].reshape(nb, BLK)
    right = edge_index[1].reshape(nb, BLK)
    out = _sc_scores(tab_left, tab_right, left, right)
    return out.reshape(num_e)
